# baseline (speedup 1.0000x reference)
import functools
import math

import jax
import jax.numpy as jnp
from jax import lax
from jax.experimental import pallas as pl
from jax.experimental.pallas import tpu as pltpu
from jax.experimental.pallas import tpu_sc as plsc

N_MOVIE = 10000
N_AUTHOR = 10000
E = 320000
D_IN = 128
D_HID = 256
N_CLS = 16

NC = 2
NS = 16
NW = NC * NS

EW1 = E // NW
C1 = 80
NCH1 = EW1 // C1

EW2 = E // NS
C2 = 80
NCH2 = EW2 // C2

DRAIN = 80
NDCHUNK = N_MOVIE // DRAIN
NDROUND = (NDCHUNK + NS - 1) // NS

ACC_W = 128


def _tc_proj_body(xm_ref, xa_ref, wq_ref, bq_ref, wk_ref, bk_ref, wv_ref,
                  bv_ref, arel_ref, mrel_ref, p_ref,
                  q_out, k_out, v_out,
                  wkf, bkf, wvf, bvf):
    @pl.when(pl.program_id(0) == 0)
    def _fold():
        wkf[...] = jnp.dot(wk_ref[...], arel_ref[...],
                           preferred_element_type=jnp.float32)
        bkf[...] = jnp.dot(bk_ref[...], arel_ref[...],
                           preferred_element_type=jnp.float32)
        wvf[...] = jnp.dot(wv_ref[...], mrel_ref[...],
                           preferred_element_type=jnp.float32)
        bvf[...] = jnp.dot(bv_ref[...], mrel_ref[...],
                           preferred_element_type=jnp.float32)

    scale = p_ref[0, 0] * (1.0 / math.sqrt(256.0))
    q_out[...] = (jnp.dot(xm_ref[...], wq_ref[...],
                          preferred_element_type=jnp.float32)
                  + bq_ref[...]) * scale
    k_out[...] = jnp.dot(xa_ref[...], wkf[...],
                         preferred_element_type=jnp.float32) + bkf[...]
    v_out[...] = jnp.dot(xa_ref[...], wvf[...],
                         preferred_element_type=jnp.float32) + bvf[...]


def _tc_proj(xm, xa, wq, bq, wk, bk, wv, bv, arel, mrel, p):
    R = 1000
    grid = (N_MOVIE // R,)
    row_spec = lambda shp: pl.BlockSpec(shp, lambda i: (i, 0))
    full_spec = lambda shp: pl.BlockSpec(shp, lambda i: (0, 0))
    return pl.pallas_call(
        _tc_proj_body,
        grid=grid,
        in_specs=[
            row_spec((R, D_IN)), row_spec((R, D_IN)),
            full_spec((D_IN, D_HID)), full_spec((1, D_HID)),
            full_spec((D_IN, D_HID)), full_spec((1, D_HID)),
            full_spec((D_IN, D_HID)), full_spec((1, D_HID)),
            full_spec((D_HID, D_HID)), full_spec((D_HID, D_HID)),
            full_spec((1, 1)),
        ],
        out_specs=[row_spec((R, D_HID))] * 3,
        out_shape=[jax.ShapeDtypeStruct((N_MOVIE, D_HID), jnp.float32)] * 3,
        scratch_shapes=[
            pltpu.VMEM((D_IN, D_HID), jnp.float32),
            pltpu.VMEM((1, D_HID), jnp.float32),
            pltpu.VMEM((D_IN, D_HID), jnp.float32),
            pltpu.VMEM((1, D_HID), jnp.float32),
        ],
        compiler_params=pltpu.CompilerParams(
            dimension_semantics=("arbitrary",)),
    )(xm, xa, wq, bq, wk, bk, wv, bv, arel, mrel, p)


_GDN = lax.GatherDimensionNumbers(
    offset_dims=(), collapsed_slice_dims=(0,), start_index_map=(0,))


def _lane_perm(v, d):
    idx = (lax.iota(jnp.int32, 16) ^ d).reshape(16, 1)
    return lax.gather(v, idx, _GDN, (1,),
                      mode=lax.GatherScatterMode.PROMISE_IN_BOUNDS)


def _lane_sum(v):
    for d in (8, 4, 2, 1):
        v = v + _lane_perm(v, d)
    return v


def _lane_max(v):
    for d in (8, 4, 2, 1):
        v = jnp.maximum(v, _lane_perm(v, d))
    return v


def _sc_pass1_body(q_hbm, k_hbm, src_hbm, dst_hbm,
                   alpha_hbm, pmax_hbm,
                   dsti, srci, qrows0, krows0, qrows1, krows1, av, pmv,
                   qs0, ks0, qs1, ks1):
    wid = lax.axis_index("s") * NC + lax.axis_index("c")
    base0 = wid * EW1

    pltpu.sync_copy(dst_hbm.at[pl.ds(base0, EW1)], dsti)
    pltpu.sync_copy(src_hbm.at[pl.ds(base0, EW1)], srci)

    def issue(i, qrows, krows, qs, ks):
        pltpu.async_copy(q_hbm.at[dsti.at[pl.ds(i * C1, C1)]], qrows, qs)
        pltpu.async_copy(k_hbm.at[srci.at[pl.ds(i * C1, C1)]], krows, ks)

    issue(0, qrows0, krows0, qs0, ks0)
    issue(1, qrows1, krows1, qs1, ks1)

    lanes = lax.iota(jnp.int32, 16)
    masks = {d: (lanes & d) == 0 for d in (8, 4, 2, 1)}
    perms = {d: (lanes ^ d).reshape(16, 1) for d in (8, 4, 2, 1)}
    bitrev = [int(f"{t:04b}"[::-1], 2) for t in range(16)]

    def compute(i, qrows, krows, m):
        def group(g, m):
            accs = []
            for t in range(16):
                e = g * 16 + bitrev[t]
                a0 = qrows[e, pl.ds(0, 16)] * krows[e, pl.ds(0, 16)]
                a1 = qrows[e, pl.ds(16, 16)] * krows[e, pl.ds(16, 16)]
                for jj in range(2, 16, 2):
                    a0 = a0 + (qrows[e, pl.ds(16 * jj, 16)]
                               * krows[e, pl.ds(16 * jj, 16)])
                    a1 = a1 + (qrows[e, pl.ds(16 * jj + 16, 16)]
                               * krows[e, pl.ds(16 * jj + 16, 16)])
                accs.append(a0 + a1)
            cur = accs
            for d in (8, 4, 2, 1):
                nxt = []
                for p in range(0, len(cur), 2):
                    fa = cur[p] + lax.gather(
                        cur[p], perms[d], _GDN, (1,),
                        mode=lax.GatherScatterMode.PROMISE_IN_BOUNDS)
                    fb = cur[p + 1] + lax.gather(
                        cur[p + 1], perms[d], _GDN, (1,),
                        mode=lax.GatherScatterMode.PROMISE_IN_BOUNDS)
                    nxt.append(jnp.where(masks[d], fa, fb))
                cur = nxt
            vec = cur[0]
            av[pl.ds(i * C1 + g * 16, 16)] = vec
            return jnp.maximum(m, vec)

        return lax.fori_loop(0, C1 // 16, group, m)

    pmv[...] = jnp.full((16,), -1e30, jnp.float32)

    def chunk(i, c):
        b = i & 1
        for bb, (qrows, krows, qs, ks) in enumerate(
                ((qrows0, krows0, qs0, ks0), (qrows1, krows1, qs1, ks1))):
            @pl.when(b == bb)
            def _():
                pltpu.make_async_copy(
                    q_hbm.at[dsti.at[pl.ds(0, C1)]], qrows, qs).wait()
                pltpu.make_async_copy(
                    k_hbm.at[srci.at[pl.ds(0, C1)]], krows, ks).wait()
                pmv[...] = compute(i, qrows, krows, pmv[...])

                @pl.when(i + 2 < NCH1)
                def _():
                    issue(i + 2, qrows, krows, qs, ks)

        return c

    lax.fori_loop(0, NCH1, chunk, 0)
    pltpu.sync_copy(av, alpha_hbm.at[pl.ds(base0, EW1)])
    pltpu.sync_copy(pmv, pmax_hbm.at[pl.ds(wid * 16, 16)])


def _sc_pass1(q, k, src, dst):
    mesh = plsc.VectorSubcoreMesh(core_axis_name="c", subcore_axis_name="s")
    f = functools.partial(
        pl.kernel,
        mesh=mesh,
        out_type=[
            jax.ShapeDtypeStruct((E,), jnp.float32),
            jax.ShapeDtypeStruct((NW * 16,), jnp.float32),
        ],
        scratch_types=[
            pltpu.VMEM((EW1,), jnp.int32),
            pltpu.VMEM((EW1,), jnp.int32),
            pltpu.VMEM((C1, D_HID), jnp.float32),
            pltpu.VMEM((C1, D_HID), jnp.float32),
            pltpu.VMEM((C1, D_HID), jnp.float32),
            pltpu.VMEM((C1, D_HID), jnp.float32),
            pltpu.VMEM((EW1,), jnp.float32),
            pltpu.VMEM((16,), jnp.float32),
            pltpu.SemaphoreType.DMA,
            pltpu.SemaphoreType.DMA,
            pltpu.SemaphoreType.DMA,
            pltpu.SemaphoreType.DMA,
        ],
    )(_sc_pass1_body)
    return f(q, k, src, dst)


DEN_ROWS = 80


def _sc_den_body(dst_hbm, alpha_hbm, pmax_hbm, den0_hbm, den1_hbm,
                 den_sh, dsta, ava, didx0, didx1, md0, md1, pmr, ds0, ds1):
    cid = lax.axis_index("c")
    sid = lax.axis_index("s")
    wid = sid * NC + cid
    base0 = wid * EW1

    pltpu.sync_copy(pmax_hbm, pmr)

    def mrow(r, mv):
        return jnp.maximum(mv, pmr[pl.ds(16 * r, 16)])

    mv = lax.fori_loop(0, NW, mrow, jnp.full((16,), -1e30, jnp.float32))
    gmax = _lane_max(mv)

    pltpu.sync_copy(dst_hbm.at[pl.ds(base0, EW1)], dsta)
    pltpu.sync_copy(alpha_hbm.at[pl.ds(base0, EW1)], ava)

    def exloop(r, c):
        ava[pl.ds(16 * r, 16)] = jnp.exp(ava[pl.ds(16 * r, 16)] - gmax)
        return c

    lax.fori_loop(0, EW1 // 16, exloop, 0)

    def zmd(r, c):
        for j in range(8):
            md0[r, pl.ds(16 * j, 16)] = jnp.zeros((16,), jnp.float32)
            md1[r, pl.ds(16 * j, 16)] = jnp.zeros((16,), jnp.float32)
        return c

    lax.fori_loop(0, C1, zmd, 0)

    @pl.when(sid == 0)
    def _():
        pltpu.sync_copy(md0, den_sh)

    plsc.subcore_barrier()

    lanes = lax.iota(jnp.int32, 16)

    def chunk(j, c):
        b = j & 1
        for bb, (md, didx, dsem) in enumerate(
                ((md0, didx0, ds0), (md1, didx1, ds1))):
            @pl.when(b == bb)
            def _():
                @pl.when(j >= 2)
                def _():
                    pltpu.make_async_copy(md, den_sh.at[didx], dsem).wait()

                def grp(g, c2):
                    eb = j * C1 + g * 16
                    dv = dsta[pl.ds(eb, 16)]
                    exv = ava[pl.ds(eb, 16)]
                    didx[pl.ds(16 * g, 16)] = lax.shift_right_logical(dv, 7)
                    ebo = jnp.maximum(eb - 2 * C1, 0)
                    dvo = dsta[pl.ds(ebo, 16)]
                    for t in range(16):
                        e = 16 * g + t
                        md[e, pl.ds(dvo[t] & 112, 16)] = jnp.zeros(
                            (16,), jnp.float32)
                        d = dv[t]
                        md[e, pl.ds(d & 112, 16)] = jnp.where(
                            lanes == (d & 15), exv[t], 0.0)
                    return c2

                lax.fori_loop(0, C1 // 16, grp, 0)
                pltpu.async_copy(md, den_sh.at[didx], dsem, add=True)

        return c

    lax.fori_loop(0, NCH1, chunk, 0)
    pltpu.make_async_copy(md0, den_sh.at[didx0], ds0).wait()
    pltpu.make_async_copy(md1, den_sh.at[didx1], ds1).wait()
    plsc.subcore_barrier()

    @pl.when(sid == 0)
    def _():
        pltpu.sync_copy(den_sh, md0)

        @pl.when(cid == 0)
        def _():
            pltpu.sync_copy(md0, den0_hbm)

        @pl.when(cid == 1)
        def _():
            pltpu.sync_copy(md0, den1_hbm)


def _sc_den(dst, alpha, pmax):
    mesh = plsc.VectorSubcoreMesh(core_axis_name="c", subcore_axis_name="s")
    f = functools.partial(
        pl.kernel,
        mesh=mesh,
        out_type=[
            jax.ShapeDtypeStruct((DEN_ROWS, 128), jnp.float32),
            jax.ShapeDtypeStruct((DEN_ROWS, 128), jnp.float32),
        ],
        scratch_types=[
            pltpu.VMEM_SHARED((DEN_ROWS, 128), jnp.float32),
            pltpu.VMEM((EW1,), jnp.int32),
            pltpu.VMEM((EW1,), jnp.float32),
            pltpu.VMEM((C1,), jnp.int32),
            pltpu.VMEM((C1,), jnp.int32),
            pltpu.VMEM((C1, 128), jnp.float32),
            pltpu.VMEM((C1, 128), jnp.float32),
            pltpu.VMEM((NW * 16,), jnp.float32),
            pltpu.SemaphoreType.DMA,
            pltpu.SemaphoreType.DMA,
        ],
    )(_sc_den_body)
    return f(dst, alpha, pmax)


def _sc_pass2_body(v_hbm, src_hbm, dst_hbm, alpha_hbm, pmax_hbm,
                   out0_hbm, out1_hbm,
                   acc_sh,
                   srci0, srci1, dsti0, dsti1, scidx0, scidx1, vidx0, vidx1,
                   av0, av1, exv0, exv1, vrows0, vrows1, msg0, msg1, pmr,
                   is0, is1, gs0, gs1, ss0, ss1):
    cid = lax.axis_index("c")
    sid = lax.axis_index("s")

    pltpu.sync_copy(pmax_hbm, pmr)

    def mrow(r, mv):
        return jnp.maximum(mv, pmr[pl.ds(16 * r, 16)])

    mv = lax.fori_loop(0, NW, mrow, jnp.full((16,), -1e30, jnp.float32))
    gmax = _lane_max(mv)

    def zmsg(r, c):
        for j in range(8):
            msg0[r, pl.ds(16 * j, 16)] = jnp.zeros((16,), jnp.float32)
        return c

    lax.fori_loop(0, C2, zmsg, 0)

    def zchunk(t, c):
        ch = sid + t * NS

        @pl.when(ch < NDCHUNK)
        def _():
            pltpu.sync_copy(msg0, acc_sh.at[pl.ds(ch * DRAIN, DRAIN)])

        return c

    lax.fori_loop(0, NDROUND, zchunk, 0)
    plsc.subcore_barrier()

    ebase = sid * EW2
    bufs = ((srci0, dsti0, scidx0, vidx0, av0, exv0, vrows0, msg0, is0, gs0,
             ss0),
            (srci1, dsti1, scidx1, vidx1, av1, exv1, vrows1, msg1, is1, gs1,
             ss1))

    def vec16(srci, vidx, av, exv):
        def body(t, c):
            s = srci[pl.ds(16 * t, 16)]
            vidx[pl.ds(16 * t, 16)] = s * 2 + cid
            exv[pl.ds(16 * t, 16)] = jnp.exp(av[pl.ds(16 * t, 16)] - gmax)
            return c

        lax.fori_loop(0, C2 // 16, body, 0)

    def cp16(src_ref, dst_ref):
        def body(t, c):
            dst_ref[pl.ds(16 * t, 16)] = src_ref[pl.ds(16 * t, 16)]
            return c

        lax.fori_loop(0, C2 // 16, body, 0)

    def issue_idx(i, srci, dsti, av, isem):
        base = ebase + i * C2
        pltpu.async_copy(src_hbm.at[pl.ds(base, C2)], srci, isem)
        pltpu.async_copy(dst_hbm.at[pl.ds(base, C2)], dsti, isem)
        pltpu.async_copy(alpha_hbm.at[pl.ds(base, C2)], av, isem)

    def wait_idx(srci, dsti, av, isem):
        pltpu.make_async_copy(src_hbm.at[pl.ds(0, C2)], srci, isem).wait()
        pltpu.make_async_copy(dst_hbm.at[pl.ds(0, C2)], dsti, isem).wait()
        pltpu.make_async_copy(alpha_hbm.at[pl.ds(0, C2)], av, isem).wait()

    pltpu.sync_copy(src_hbm.at[pl.ds(ebase, C2)], srci0)
    pltpu.sync_copy(dst_hbm.at[pl.ds(ebase, C2)], dsti0)
    pltpu.sync_copy(alpha_hbm.at[pl.ds(ebase, C2)], av0)
    vec16(srci0, vidx0, av0, exv0)
    pltpu.async_copy(v_hbm.at[vidx0], vrows0, gs0)
    issue_idx(1, srci1, dsti1, av1, is1)

    def chunk(i, c):
        b = i & 1
        for bb in (0, 1):
            (srci, dsti, scidx, vidx, av, exv, vrows, msg, isem, gsem,
             ssem) = bufs[bb]
            (srcn, dstn, scidxn, vidxn, avn, exvn, vrowsn, msgn, isemn,
             gsemn, ssemn) = bufs[1 - bb]

            @pl.when(b == bb)
            def _():
                @pl.when(i + 1 < NCH2)
                def _():
                    wait_idx(srcn, dstn, avn, isemn)
                    vec16(srcn, vidxn, avn, exvn)
                    pltpu.async_copy(v_hbm.at[vidxn], vrowsn, gsemn)

                pltpu.make_async_copy(v_hbm.at[vidx], vrows, gsem).wait()

                def group(g, c2):
                    exq = exv[pl.ds(g * 16, 16)]
                    for t in range(16):
                        e = g * 16 + t
                        ex = exq[t]
                        for j in range(8):
                            msg[e, pl.ds(16 * j, 16)] = (
                                vrows[e, pl.ds(16 * j, 16)] * ex)
                    return c2

                lax.fori_loop(0, C2 // 16, group, 0)

                @pl.when(i >= 1)
                def _():
                    pltpu.make_async_copy(
                        msgn, acc_sh.at[scidxn], ssemn).wait()

                cp16(dsti, scidx)

                @pl.when(i + 2 < NCH2)
                def _():
                    issue_idx(i + 2, srci, dsti, av, isem)

                pltpu.async_copy(msg, acc_sh.at[scidx], ssem, add=True)

        return c

    lax.fori_loop(0, NCH2, chunk, 0)
    pltpu.make_async_copy(msg1, acc_sh.at[scidx1], ss1).wait()
    plsc.subcore_barrier()

    def drain(t, c):
        ch = sid + t * NS

        @pl.when(ch < NDCHUNK)
        def _():
            r = ch * DRAIN
            pltpu.sync_copy(acc_sh.at[pl.ds(r, DRAIN)], msg0)

            @pl.when(cid == 0)
            def _():
                pltpu.sync_copy(msg0, out0_hbm.at[pl.ds(r, DRAIN)])

            @pl.when(cid == 1)
            def _():
                pltpu.sync_copy(msg0, out1_hbm.at[pl.ds(r, DRAIN)])

        return c

    lax.fori_loop(0, NDROUND, drain, 0)


def _sc_pass2(v2, src, dst, alpha, pmax):
    mesh = plsc.VectorSubcoreMesh(core_axis_name="c", subcore_axis_name="s")
    f = functools.partial(
        pl.kernel,
        mesh=mesh,
        out_type=[
            jax.ShapeDtypeStruct((N_MOVIE, ACC_W), jnp.float32),
            jax.ShapeDtypeStruct((N_MOVIE, ACC_W), jnp.float32),
        ],
        scratch_types=[
            pltpu.VMEM_SHARED((N_MOVIE, ACC_W), jnp.float32),
            pltpu.VMEM((C2,), jnp.int32),
            pltpu.VMEM((C2,), jnp.int32),
            pltpu.VMEM((C2,), jnp.int32),
            pltpu.VMEM((C2,), jnp.int32),
            pltpu.VMEM((C2,), jnp.int32),
            pltpu.VMEM((C2,), jnp.int32),
            pltpu.VMEM((C2,), jnp.int32),
            pltpu.VMEM((C2,), jnp.int32),
            pltpu.VMEM((C2,), jnp.float32),
            pltpu.VMEM((C2,), jnp.float32),
            pltpu.VMEM((C2,), jnp.float32),
            pltpu.VMEM((C2,), jnp.float32),
            pltpu.VMEM((C2, 128), jnp.float32),
            pltpu.VMEM((C2, 128), jnp.float32),
            pltpu.VMEM((C2, ACC_W), jnp.float32),
            pltpu.VMEM((C2, ACC_W), jnp.float32),
            pltpu.VMEM((NW * 16,), jnp.float32),
            pltpu.SemaphoreType.DMA,
            pltpu.SemaphoreType.DMA,
            pltpu.SemaphoreType.DMA,
            pltpu.SemaphoreType.DMA,
            pltpu.SemaphoreType.DMA,
            pltpu.SemaphoreType.DMA,
        ],
    )(_sc_pass2_body)
    return f(v2, src, dst, alpha, pmax)


def _tc_final_body(o0_ref, o1_ref, d0_ref, d1_ref, wa_ref, ba_ref, wlin_ref,
                   blin_ref, y_ref, walf, balf):
    @pl.when(pl.program_id(0) == 0)
    def _fold():
        walf[...] = jnp.dot(wa_ref[...], wlin_ref[...],
                            preferred_element_type=jnp.float32)
        balf[...] = jnp.dot(ba_ref[...], wlin_ref[...],
                            preferred_element_type=jnp.float32) + blin_ref[...]

    agg = jnp.concatenate([o0_ref[...], o1_ref[...]], axis=1)
    den = d0_ref[...] + d1_ref[...] + 1e-16
    h = jax.nn.gelu(agg / den)
    y_ref[...] = jnp.dot(h, walf[...],
                         preferred_element_type=jnp.float32) + balf[...]


def _tc_final(o0, o1, d0, d1, wa, ba, wlin, blin):
    R = 1000
    grid = (N_MOVIE // R,)
    return pl.pallas_call(
        _tc_final_body,
        grid=grid,
        in_specs=[
            pl.BlockSpec((R, ACC_W), lambda i: (i, 0)),
            pl.BlockSpec((R, ACC_W), lambda i: (i, 0)),
            pl.BlockSpec((R, 1), lambda i: (i, 0)),
            pl.BlockSpec((R, 1), lambda i: (i, 0)),
            pl.BlockSpec((D_HID, D_HID), lambda i: (0, 0)),
            pl.BlockSpec((1, D_HID), lambda i: (0, 0)),
            pl.BlockSpec((D_HID, N_CLS), lambda i: (0, 0)),
            pl.BlockSpec((1, N_CLS), lambda i: (0, 0)),
        ],
        out_specs=pl.BlockSpec((R, N_CLS), lambda i: (i, 0)),
        out_shape=jax.ShapeDtypeStruct((N_MOVIE, N_CLS), jnp.float32),
        scratch_shapes=[
            pltpu.VMEM((D_HID, N_CLS), jnp.float32),
            pltpu.VMEM((1, N_CLS), jnp.float32),
        ],
        compiler_params=pltpu.CompilerParams(
            dimension_semantics=("arbitrary",)),
    )(o0, o1, d0, d1, wa, ba, wlin, blin)


def kernel(x_movie, x_author, Wk, bk, Wq, bq, Wv, bv, a_rel, m_rel, p_rel,
           Wa, ba, Wlin, blin, edge_index):
    ei = edge_index.astype(jnp.int32)
    src = ei[0]
    dst = ei[1]
    arel = a_rel.reshape(D_HID, D_HID)
    mrel = m_rel.reshape(D_HID, D_HID)

    q, k, v = _tc_proj(
        x_movie, x_author,
        Wq, bq.reshape(1, D_HID),
        Wk, bk.reshape(1, D_HID),
        Wv, bv.reshape(1, D_HID),
        arel, mrel, p_rel.reshape(1, 1).astype(jnp.float32))

    alpha, pmax = _sc_pass1(q, k, src, dst)
    den0, den1 = _sc_den(dst, alpha, pmax)

    v2 = v.reshape(2 * N_MOVIE, 128)
    o0, o1 = _sc_pass2(v2, src, dst, alpha, pmax)

    d0 = den0.reshape(DEN_ROWS * 128)[:N_MOVIE].reshape(N_MOVIE, 1)
    d1 = den1.reshape(DEN_ROWS * 128)[:N_MOVIE].reshape(N_MOVIE, 1)
    return _tc_final(o0, o1, d0, d1,
                     Wa, ba.reshape(1, D_HID),
                     Wlin, blin.reshape(1, N_CLS))


# speedup vs baseline: 1.3598x; 1.3598x over previous
import functools
import math

import jax
import jax.numpy as jnp
from jax import lax
from jax.experimental import pallas as pl
from jax.experimental.pallas import tpu as pltpu
from jax.experimental.pallas import tpu_sc as plsc

N_MOVIE = 10000
N_AUTHOR = 10000
E = 320000
D_IN = 128
D_HID = 256
N_CLS = 16

NC = 2
NS = 16
NW = NC * NS

EW1 = E // NW
C1 = 80
NCH1 = EW1 // C1

EW2 = E // NS
C2 = 80
NCH2 = EW2 // C2

DRAIN = 80
NDCHUNK = N_MOVIE // DRAIN
NDROUND = (NDCHUNK + NS - 1) // NS

ACC_W = 128


def _tc_proj_body(xm_ref, xa_ref, g_ref, delta_ref, wv_ref, bv_ref, mrel_ref,
                  q_out, k_out, v_out,
                  wvf, bvf):
    @pl.when(pl.program_id(0) == 0)
    def _fold():
        wvf[...] = jnp.dot(wv_ref[...], mrel_ref[...],
                           preferred_element_type=jnp.float32)
        bvf[...] = jnp.dot(bv_ref[...], mrel_ref[...],
                           preferred_element_type=jnp.float32)

    q_out[...] = xm_ref[...] + delta_ref[...]
    k_out[...] = lax.dot_general(
        xa_ref[...], g_ref[...], (((1,), (1,)), ((), ())),
        preferred_element_type=jnp.float32)
    v_out[...] = jnp.dot(xa_ref[...], wvf[...],
                         preferred_element_type=jnp.float32) + bvf[...]


def _tc_proj(xm, xa, g, delta, wv, bv, mrel):
    R = 1000
    grid = (N_MOVIE // R,)
    row_spec = lambda shp: pl.BlockSpec(shp, lambda i: (i, 0))
    full_spec = lambda shp: pl.BlockSpec(shp, lambda i: (0, 0))
    return pl.pallas_call(
        _tc_proj_body,
        grid=grid,
        in_specs=[
            row_spec((R, D_IN)), row_spec((R, D_IN)),
            full_spec((D_IN, D_IN)), full_spec((1, D_IN)),
            full_spec((D_IN, D_HID)), full_spec((1, D_HID)),
            full_spec((D_HID, D_HID)),
        ],
        out_specs=[row_spec((R, D_IN)), row_spec((R, D_IN)),
                   row_spec((R, D_HID))],
        out_shape=[
            jax.ShapeDtypeStruct((N_MOVIE, D_IN), jnp.float32),
            jax.ShapeDtypeStruct((N_MOVIE, D_IN), jnp.float32),
            jax.ShapeDtypeStruct((N_MOVIE, D_HID), jnp.float32),
        ],
        scratch_shapes=[
            pltpu.VMEM((D_IN, D_HID), jnp.float32),
            pltpu.VMEM((1, D_HID), jnp.float32),
        ],
        compiler_params=pltpu.CompilerParams(
            dimension_semantics=("arbitrary",)),
    )(xm, xa, g, delta, wv, bv, mrel)


_GDN = lax.GatherDimensionNumbers(
    offset_dims=(), collapsed_slice_dims=(0,), start_index_map=(0,))


def _lane_perm(v, d):
    idx = (lax.iota(jnp.int32, 16) ^ d).reshape(16, 1)
    return lax.gather(v, idx, _GDN, (1,),
                      mode=lax.GatherScatterMode.PROMISE_IN_BOUNDS)


def _lane_sum(v):
    for d in (8, 4, 2, 1):
        v = v + _lane_perm(v, d)
    return v


def _lane_max(v):
    for d in (8, 4, 2, 1):
        v = jnp.maximum(v, _lane_perm(v, d))
    return v


def _sc_pass1_body(q_hbm, k_hbm, src_hbm, dst_hbm,
                   alpha_hbm, pmax_hbm,
                   dsti, srci, qrows0, krows0, qrows1, krows1, av, pmv,
                   qs0, ks0, qs1, ks1):
    wid = lax.axis_index("s") * NC + lax.axis_index("c")
    base0 = wid * EW1

    pltpu.sync_copy(dst_hbm.at[pl.ds(base0, EW1)], dsti)
    pltpu.sync_copy(src_hbm.at[pl.ds(base0, EW1)], srci)

    def issue(i, qrows, krows, qs, ks):
        pltpu.async_copy(q_hbm.at[dsti.at[pl.ds(i * C1, C1)]], qrows, qs)
        pltpu.async_copy(k_hbm.at[srci.at[pl.ds(i * C1, C1)]], krows, ks)

    issue(0, qrows0, krows0, qs0, ks0)
    issue(1, qrows1, krows1, qs1, ks1)

    lanes = lax.iota(jnp.int32, 16)

    def compute(i, qrows, krows, m):
        def group(g, m):
            vec = jnp.zeros((16,), jnp.float32)
            for t in range(16):
                e = g * 16 + t
                a0 = qrows[e, pl.ds(0, 16)] * krows[e, pl.ds(0, 16)]
                a1 = qrows[e, pl.ds(16, 16)] * krows[e, pl.ds(16, 16)]
                for jj in range(2, 8, 2):
                    a0 = a0 + (qrows[e, pl.ds(16 * jj, 16)]
                               * krows[e, pl.ds(16 * jj, 16)])
                    a1 = a1 + (qrows[e, pl.ds(16 * jj + 16, 16)]
                               * krows[e, pl.ds(16 * jj + 16, 16)])
                s = _lane_sum(a0 + a1)
                vec = jnp.where(lanes == t, s, vec)
                m = jnp.maximum(m, s)
            av[pl.ds(i * C1 + g * 16, 16)] = vec
            return m

        return lax.fori_loop(0, C1 // 16, group, m)

    pmv[...] = jnp.full((16,), -1e30, jnp.float32)

    def chunk(i, c):
        b = i & 1
        for bb, (qrows, krows, qs, ks) in enumerate(
                ((qrows0, krows0, qs0, ks0), (qrows1, krows1, qs1, ks1))):
            @pl.when(b == bb)
            def _():
                pltpu.make_async_copy(
                    q_hbm.at[dsti.at[pl.ds(0, C1)]], qrows, qs).wait()
                pltpu.make_async_copy(
                    k_hbm.at[srci.at[pl.ds(0, C1)]], krows, ks).wait()
                pmv[...] = compute(i, qrows, krows, pmv[...])

                @pl.when(i + 2 < NCH1)
                def _():
                    issue(i + 2, qrows, krows, qs, ks)

        return c

    lax.fori_loop(0, NCH1, chunk, 0)
    pltpu.sync_copy(av, alpha_hbm.at[pl.ds(base0, EW1)])
    pltpu.sync_copy(pmv, pmax_hbm.at[pl.ds(wid * 16, 16)])


def _sc_pass1(q, k, src, dst):
    mesh = plsc.VectorSubcoreMesh(core_axis_name="c", subcore_axis_name="s")
    f = functools.partial(
        pl.kernel,
        mesh=mesh,
        out_type=[
            jax.ShapeDtypeStruct((E,), jnp.float32),
            jax.ShapeDtypeStruct((NW * 16,), jnp.float32),
        ],
        scratch_types=[
            pltpu.VMEM((EW1,), jnp.int32),
            pltpu.VMEM((EW1,), jnp.int32),
            pltpu.VMEM((C1, D_IN), jnp.float32),
            pltpu.VMEM((C1, D_IN), jnp.float32),
            pltpu.VMEM((C1, D_IN), jnp.float32),
            pltpu.VMEM((C1, D_IN), jnp.float32),
            pltpu.VMEM((EW1,), jnp.float32),
            pltpu.VMEM((16,), jnp.float32),
            pltpu.SemaphoreType.DMA,
            pltpu.SemaphoreType.DMA,
            pltpu.SemaphoreType.DMA,
            pltpu.SemaphoreType.DMA,
        ],
    )(_sc_pass1_body)
    return f(q, k, src, dst)


DEN_ROWS = 80


def _sc_den_body(dst_hbm, alpha_hbm, pmax_hbm, den0_hbm, den1_hbm,
                 den_sh, dsta, ava, didx0, didx1, md0, md1, pmr, ds0, ds1):
    cid = lax.axis_index("c")
    sid = lax.axis_index("s")
    wid = sid * NC + cid
    base0 = wid * EW1

    pltpu.sync_copy(pmax_hbm, pmr)

    def mrow(r, mv):
        return jnp.maximum(mv, pmr[pl.ds(16 * r, 16)])

    mv = lax.fori_loop(0, NW, mrow, jnp.full((16,), -1e30, jnp.float32))
    gmax = _lane_max(mv)

    pltpu.sync_copy(dst_hbm.at[pl.ds(base0, EW1)], dsta)
    pltpu.sync_copy(alpha_hbm.at[pl.ds(base0, EW1)], ava)

    def exloop(r, c):
        ava[pl.ds(16 * r, 16)] = jnp.exp(ava[pl.ds(16 * r, 16)] - gmax)
        return c

    lax.fori_loop(0, EW1 // 16, exloop, 0)

    def zmd(r, c):
        for j in range(8):
            md0[r, pl.ds(16 * j, 16)] = jnp.zeros((16,), jnp.float32)
            md1[r, pl.ds(16 * j, 16)] = jnp.zeros((16,), jnp.float32)
        return c

    lax.fori_loop(0, C1, zmd, 0)

    @pl.when(sid == 0)
    def _():
        pltpu.sync_copy(md0, den_sh)

    plsc.subcore_barrier()

    lanes = lax.iota(jnp.int32, 16)

    def chunk(j, c):
        b = j & 1
        for bb, (md, didx, dsem) in enumerate(
                ((md0, didx0, ds0), (md1, didx1, ds1))):
            @pl.when(b == bb)
            def _():
                @pl.when(j >= 2)
                def _():
                    pltpu.make_async_copy(md, den_sh.at[didx], dsem).wait()

                def grp(g, c2):
                    eb = j * C1 + g * 16
                    dv = dsta[pl.ds(eb, 16)]
                    exv = ava[pl.ds(eb, 16)]
                    didx[pl.ds(16 * g, 16)] = lax.shift_right_logical(dv, 7)
                    ebo = jnp.maximum(eb - 2 * C1, 0)
                    dvo = dsta[pl.ds(ebo, 16)]
                    for t in range(16):
                        e = 16 * g + t
                        md[e, pl.ds(dvo[t] & 112, 16)] = jnp.zeros(
                            (16,), jnp.float32)
                        d = dv[t]
                        md[e, pl.ds(d & 112, 16)] = jnp.where(
                            lanes == (d & 15), exv[t], 0.0)
                    return c2

                lax.fori_loop(0, C1 // 16, grp, 0)
                pltpu.async_copy(md, den_sh.at[didx], dsem, add=True)

        return c

    lax.fori_loop(0, NCH1, chunk, 0)
    pltpu.make_async_copy(md0, den_sh.at[didx0], ds0).wait()
    pltpu.make_async_copy(md1, den_sh.at[didx1], ds1).wait()
    plsc.subcore_barrier()

    @pl.when(sid == 0)
    def _():
        pltpu.sync_copy(den_sh, md0)

        @pl.when(cid == 0)
        def _():
            pltpu.sync_copy(md0, den0_hbm)

        @pl.when(cid == 1)
        def _():
            pltpu.sync_copy(md0, den1_hbm)


def _sc_den(dst, alpha, pmax):
    mesh = plsc.VectorSubcoreMesh(core_axis_name="c", subcore_axis_name="s")
    f = functools.partial(
        pl.kernel,
        mesh=mesh,
        out_type=[
            jax.ShapeDtypeStruct((DEN_ROWS, 128), jnp.float32),
            jax.ShapeDtypeStruct((DEN_ROWS, 128), jnp.float32),
        ],
        scratch_types=[
            pltpu.VMEM_SHARED((DEN_ROWS, 128), jnp.float32),
            pltpu.VMEM((EW1,), jnp.int32),
            pltpu.VMEM((EW1,), jnp.float32),
            pltpu.VMEM((C1,), jnp.int32),
            pltpu.VMEM((C1,), jnp.int32),
            pltpu.VMEM((C1, 128), jnp.float32),
            pltpu.VMEM((C1, 128), jnp.float32),
            pltpu.VMEM((NW * 16,), jnp.float32),
            pltpu.SemaphoreType.DMA,
            pltpu.SemaphoreType.DMA,
        ],
    )(_sc_den_body)
    return f(dst, alpha, pmax)


def _sc_pass2_body(v_hbm, src_hbm, dst_hbm, alpha_hbm, pmax_hbm,
                   out0_hbm, out1_hbm,
                   acc_sh,
                   srci0, srci1, dsti0, dsti1, scidx0, scidx1, vidx0, vidx1,
                   av0, av1, exv0, exv1, vrows0, vrows1, msg0, msg1, pmr,
                   is0, is1, gs0, gs1, ss0, ss1):
    cid = lax.axis_index("c")
    sid = lax.axis_index("s")

    pltpu.sync_copy(pmax_hbm, pmr)

    def mrow(r, mv):
        return jnp.maximum(mv, pmr[pl.ds(16 * r, 16)])

    mv = lax.fori_loop(0, NW, mrow, jnp.full((16,), -1e30, jnp.float32))
    gmax = _lane_max(mv)

    def zmsg(r, c):
        for j in range(8):
            msg0[r, pl.ds(16 * j, 16)] = jnp.zeros((16,), jnp.float32)
        return c

    lax.fori_loop(0, C2, zmsg, 0)

    def zchunk(t, c):
        ch = sid + t * NS

        @pl.when(ch < NDCHUNK)
        def _():
            pltpu.sync_copy(msg0, acc_sh.at[pl.ds(ch * DRAIN, DRAIN)])

        return c

    lax.fori_loop(0, NDROUND, zchunk, 0)
    plsc.subcore_barrier()

    ebase = sid * EW2
    bufs = ((srci0, dsti0, scidx0, vidx0, av0, exv0, vrows0, msg0, is0, gs0,
             ss0),
            (srci1, dsti1, scidx1, vidx1, av1, exv1, vrows1, msg1, is1, gs1,
             ss1))

    def vec16(srci, vidx, av, exv):
        def body(t, c):
            s = srci[pl.ds(16 * t, 16)]
            vidx[pl.ds(16 * t, 16)] = s * 2 + cid
            exv[pl.ds(16 * t, 16)] = jnp.exp(av[pl.ds(16 * t, 16)] - gmax)
            return c

        lax.fori_loop(0, C2 // 16, body, 0)

    def cp16(src_ref, dst_ref):
        def body(t, c):
            dst_ref[pl.ds(16 * t, 16)] = src_ref[pl.ds(16 * t, 16)]
            return c

        lax.fori_loop(0, C2 // 16, body, 0)

    def issue_idx(i, srci, dsti, av, isem):
        base = ebase + i * C2
        pltpu.async_copy(src_hbm.at[pl.ds(base, C2)], srci, isem)
        pltpu.async_copy(dst_hbm.at[pl.ds(base, C2)], dsti, isem)
        pltpu.async_copy(alpha_hbm.at[pl.ds(base, C2)], av, isem)

    def wait_idx(srci, dsti, av, isem):
        pltpu.make_async_copy(src_hbm.at[pl.ds(0, C2)], srci, isem).wait()
        pltpu.make_async_copy(dst_hbm.at[pl.ds(0, C2)], dsti, isem).wait()
        pltpu.make_async_copy(alpha_hbm.at[pl.ds(0, C2)], av, isem).wait()

    pltpu.sync_copy(src_hbm.at[pl.ds(ebase, C2)], srci0)
    pltpu.sync_copy(dst_hbm.at[pl.ds(ebase, C2)], dsti0)
    pltpu.sync_copy(alpha_hbm.at[pl.ds(ebase, C2)], av0)
    vec16(srci0, vidx0, av0, exv0)
    pltpu.async_copy(v_hbm.at[vidx0], vrows0, gs0)
    issue_idx(1, srci1, dsti1, av1, is1)

    def chunk(i, c):
        b = i & 1
        for bb in (0, 1):
            (srci, dsti, scidx, vidx, av, exv, vrows, msg, isem, gsem,
             ssem) = bufs[bb]
            (srcn, dstn, scidxn, vidxn, avn, exvn, vrowsn, msgn, isemn,
             gsemn, ssemn) = bufs[1 - bb]

            @pl.when(b == bb)
            def _():
                @pl.when(i + 1 < NCH2)
                def _():
                    wait_idx(srcn, dstn, avn, isemn)
                    vec16(srcn, vidxn, avn, exvn)
                    pltpu.async_copy(v_hbm.at[vidxn], vrowsn, gsemn)

                cp16(dsti, scidx)

                @pl.when(i + 2 < NCH2)
                def _():
                    issue_idx(i + 2, srci, dsti, av, isem)

                pltpu.make_async_copy(v_hbm.at[vidx], vrows, gsem).wait()

                def group(g, c2):
                    exq = exv[pl.ds(g * 16, 16)]
                    for t in range(16):
                        e = g * 16 + t
                        ex = exq[t]
                        for j in range(8):
                            msg[e, pl.ds(16 * j, 16)] = (
                                vrows[e, pl.ds(16 * j, 16)] * ex)
                    return c2

                lax.fori_loop(0, C2 // 16, group, 0)
                pltpu.sync_copy(msg, acc_sh.at[scidx], add=True)

        return c

    lax.fori_loop(0, NCH2, chunk, 0)
    plsc.subcore_barrier()

    def drain(t, c):
        ch = sid + t * NS

        @pl.when(ch < NDCHUNK)
        def _():
            r = ch * DRAIN
            pltpu.sync_copy(acc_sh.at[pl.ds(r, DRAIN)], msg0)

            @pl.when(cid == 0)
            def _():
                pltpu.sync_copy(msg0, out0_hbm.at[pl.ds(r, DRAIN)])

            @pl.when(cid == 1)
            def _():
                pltpu.sync_copy(msg0, out1_hbm.at[pl.ds(r, DRAIN)])

        return c

    lax.fori_loop(0, NDROUND, drain, 0)


def _sc_pass2(v2, src, dst, alpha, pmax):
    mesh = plsc.VectorSubcoreMesh(core_axis_name="c", subcore_axis_name="s")
    f = functools.partial(
        pl.kernel,
        mesh=mesh,
        out_type=[
            jax.ShapeDtypeStruct((N_MOVIE, ACC_W), jnp.float32),
            jax.ShapeDtypeStruct((N_MOVIE, ACC_W), jnp.float32),
        ],
        scratch_types=[
            pltpu.VMEM_SHARED((N_MOVIE, ACC_W), jnp.float32),
            pltpu.VMEM((C2,), jnp.int32),
            pltpu.VMEM((C2,), jnp.int32),
            pltpu.VMEM((C2,), jnp.int32),
            pltpu.VMEM((C2,), jnp.int32),
            pltpu.VMEM((C2,), jnp.int32),
            pltpu.VMEM((C2,), jnp.int32),
            pltpu.VMEM((C2,), jnp.int32),
            pltpu.VMEM((C2,), jnp.int32),
            pltpu.VMEM((C2,), jnp.float32),
            pltpu.VMEM((C2,), jnp.float32),
            pltpu.VMEM((C2,), jnp.float32),
            pltpu.VMEM((C2,), jnp.float32),
            pltpu.VMEM((C2, 128), jnp.float32),
            pltpu.VMEM((C2, 128), jnp.float32),
            pltpu.VMEM((C2, ACC_W), jnp.float32),
            pltpu.VMEM((C2, ACC_W), jnp.float32),
            pltpu.VMEM((NW * 16,), jnp.float32),
            pltpu.SemaphoreType.DMA,
            pltpu.SemaphoreType.DMA,
            pltpu.SemaphoreType.DMA,
            pltpu.SemaphoreType.DMA,
            pltpu.SemaphoreType.DMA,
            pltpu.SemaphoreType.DMA,
        ],
    )(_sc_pass2_body)
    return f(v2, src, dst, alpha, pmax)


def _tc_final_body(o0_ref, o1_ref, d0_ref, d1_ref, wa_ref, ba_ref, wlin_ref,
                   blin_ref, y_ref, walf, balf):
    @pl.when(pl.program_id(0) == 0)
    def _fold():
        walf[...] = jnp.dot(wa_ref[...], wlin_ref[...],
                            preferred_element_type=jnp.float32)
        balf[...] = jnp.dot(ba_ref[...], wlin_ref[...],
                            preferred_element_type=jnp.float32) + blin_ref[...]

    agg = jnp.concatenate([o0_ref[...], o1_ref[...]], axis=1)
    den = d0_ref[...] + d1_ref[...] + 1e-16
    h = jax.nn.gelu(agg / den)
    y_ref[...] = jnp.dot(h, walf[...],
                         preferred_element_type=jnp.float32) + balf[...]


def _tc_final(o0, o1, d0, d1, wa, ba, wlin, blin):
    R = 1000
    grid = (N_MOVIE // R,)
    return pl.pallas_call(
        _tc_final_body,
        grid=grid,
        in_specs=[
            pl.BlockSpec((R, ACC_W), lambda i: (i, 0)),
            pl.BlockSpec((R, ACC_W), lambda i: (i, 0)),
            pl.BlockSpec((R, 1), lambda i: (i, 0)),
            pl.BlockSpec((R, 1), lambda i: (i, 0)),
            pl.BlockSpec((D_HID, D_HID), lambda i: (0, 0)),
            pl.BlockSpec((1, D_HID), lambda i: (0, 0)),
            pl.BlockSpec((D_HID, N_CLS), lambda i: (0, 0)),
            pl.BlockSpec((1, N_CLS), lambda i: (0, 0)),
        ],
        out_specs=pl.BlockSpec((R, N_CLS), lambda i: (i, 0)),
        out_shape=jax.ShapeDtypeStruct((N_MOVIE, N_CLS), jnp.float32),
        scratch_shapes=[
            pltpu.VMEM((D_HID, N_CLS), jnp.float32),
            pltpu.VMEM((1, N_CLS), jnp.float32),
        ],
        compiler_params=pltpu.CompilerParams(
            dimension_semantics=("arbitrary",)),
    )(o0, o1, d0, d1, wa, ba, wlin, blin)


def kernel(x_movie, x_author, Wk, bk, Wq, bq, Wv, bv, a_rel, m_rel, p_rel,
           Wa, ba, Wlin, blin, edge_index):
    ei = edge_index.astype(jnp.int32)
    src = ei[0]
    dst = ei[1]
    arel = a_rel.reshape(D_HID, D_HID)
    mrel = m_rel.reshape(D_HID, D_HID)

    s = p_rel.reshape(-1)[0].astype(jnp.float32) * (1.0 / 16.0)
    w2 = Wk @ arel
    g = (s * Wq) @ w2.T
    w = w2 @ (bq * s)
    dsolve = jnp.linalg.solve(g.T, w)
    delta = jnp.where(jnp.any(w != 0), dsolve, jnp.zeros_like(w))

    q, k, v = _tc_proj(x_movie, x_author, g, delta.reshape(1, D_IN),
                       Wv, bv.reshape(1, D_HID), mrel)

    alpha, pmax = _sc_pass1(q, k, src, dst)
    den0, den1 = _sc_den(dst, alpha, pmax)

    v2 = v.reshape(2 * N_MOVIE, 128)
    o0, o1 = _sc_pass2(v2, src, dst, alpha, pmax)

    d0 = den0.reshape(DEN_ROWS * 128)[:N_MOVIE].reshape(N_MOVIE, 1)
    d1 = den1.reshape(DEN_ROWS * 128)[:N_MOVIE].reshape(N_MOVIE, 1)
    return _tc_final(o0, o1, d0, d1,
                     Wa, ba.reshape(1, D_HID),
                     Wlin, blin.reshape(1, N_CLS))


# speedup vs baseline: 1.3603x; 1.0003x over previous
import functools

import jax
import jax.numpy as jnp
from jax import lax
from jax.experimental import pallas as pl
from jax.experimental.pallas import tpu as pltpu
from jax.experimental.pallas import tpu_sc as plsc

N_MOVIE = 10000
N_AUTHOR = 10000
E = 320000
D_IN = 128
D_HID = 256
N_CLS = 16

NC = 2
NS = 16
NW = NC * NS

EW1 = E // NW
C1 = 80
NCH1 = EW1 // C1

EW2 = E // NS
C2 = 80
NCH2 = EW2 // C2

DRAIN = 80
NDCHUNK = N_MOVIE // DRAIN
NDROUND = (NDCHUNK + NS - 1) // NS

ACC_W = 128


def _tc_proj_body(xm_ref, xa_ref, g_ref, delta_ref, wv_ref, bv_ref, mrel_ref,
                  q_out, k_out, v_out,
                  wvf, bvf):
    @pl.when(pl.program_id(0) == 0)
    def _fold():
        wvf[...] = jnp.dot(wv_ref[...], mrel_ref[...],
                           preferred_element_type=jnp.float32)
        bvf[...] = jnp.dot(bv_ref[...], mrel_ref[...],
                           preferred_element_type=jnp.float32)

    q_out[...] = xm_ref[...] + delta_ref[...]
    k_out[...] = lax.dot_general(
        xa_ref[...], g_ref[...], (((1,), (1,)), ((), ())),
        preferred_element_type=jnp.float32)
    v_out[...] = jnp.dot(xa_ref[...], wvf[...],
                         preferred_element_type=jnp.float32) + bvf[...]


def _tc_proj(xm, xa, g, delta, wv, bv, mrel):
    R = 1000
    grid = (N_MOVIE // R,)
    row_spec = lambda shp: pl.BlockSpec(shp, lambda i: (i, 0))
    full_spec = lambda shp: pl.BlockSpec(shp, lambda i: (0, 0))
    return pl.pallas_call(
        _tc_proj_body,
        grid=grid,
        in_specs=[
            row_spec((R, D_IN)), row_spec((R, D_IN)),
            full_spec((D_IN, D_IN)), full_spec((1, D_IN)),
            full_spec((D_IN, D_HID)), full_spec((1, D_HID)),
            full_spec((D_HID, D_HID)),
        ],
        out_specs=[row_spec((R, D_IN)), row_spec((R, D_IN)),
                   row_spec((R, D_HID))],
        out_shape=[
            jax.ShapeDtypeStruct((N_MOVIE, D_IN), jnp.float32),
            jax.ShapeDtypeStruct((N_MOVIE, D_IN), jnp.float32),
            jax.ShapeDtypeStruct((N_MOVIE, D_HID), jnp.float32),
        ],
        scratch_shapes=[
            pltpu.VMEM((D_IN, D_HID), jnp.float32),
            pltpu.VMEM((1, D_HID), jnp.float32),
        ],
        compiler_params=pltpu.CompilerParams(
            dimension_semantics=("arbitrary",)),
    )(xm, xa, g, delta, wv, bv, mrel)


_GDN = lax.GatherDimensionNumbers(
    offset_dims=(), collapsed_slice_dims=(0,), start_index_map=(0,))


def _lane_perm(v, d):
    idx = (lax.iota(jnp.int32, 16) ^ d).reshape(16, 1)
    return lax.gather(v, idx, _GDN, (1,),
                      mode=lax.GatherScatterMode.PROMISE_IN_BOUNDS)


def _lane_sum(v):
    for d in (8, 4, 2, 1):
        v = v + _lane_perm(v, d)
    return v


def _lane_max(v):
    for d in (8, 4, 2, 1):
        v = jnp.maximum(v, _lane_perm(v, d))
    return v


def _sc_pass1_body(q_hbm, k_hbm, src_hbm, dst_hbm,
                   alpha_hbm, pmax_hbm,
                   dsti, srci, qrows0, krows0, qrows1, krows1, av, pmv,
                   qs0, ks0, qs1, ks1):
    wid = lax.axis_index("s") * NC + lax.axis_index("c")
    base0 = wid * EW1

    pltpu.sync_copy(dst_hbm.at[pl.ds(base0, EW1)], dsti)
    pltpu.sync_copy(src_hbm.at[pl.ds(base0, EW1)], srci)

    def issue(i, qrows, krows, qs, ks):
        pltpu.async_copy(q_hbm.at[dsti.at[pl.ds(i * C1, C1)]], qrows, qs)
        pltpu.async_copy(k_hbm.at[srci.at[pl.ds(i * C1, C1)]], krows, ks)

    issue(0, qrows0, krows0, qs0, ks0)
    issue(1, qrows1, krows1, qs1, ks1)

    lanes = lax.iota(jnp.int32, 16)

    def compute(i, qrows, krows, m):
        def group(g, m):
            vec = jnp.zeros((16,), jnp.float32)
            for t in range(16):
                e = g * 16 + t
                a0 = qrows[e, pl.ds(0, 16)] * krows[e, pl.ds(0, 16)]
                a1 = qrows[e, pl.ds(16, 16)] * krows[e, pl.ds(16, 16)]
                for jj in range(2, 8, 2):
                    a0 = a0 + (qrows[e, pl.ds(16 * jj, 16)]
                               * krows[e, pl.ds(16 * jj, 16)])
                    a1 = a1 + (qrows[e, pl.ds(16 * jj + 16, 16)]
                               * krows[e, pl.ds(16 * jj + 16, 16)])
                s = _lane_sum(a0 + a1)
                vec = jnp.where(lanes == t, s, vec)
                m = jnp.maximum(m, s)
            av[pl.ds(i * C1 + g * 16, 16)] = vec
            return m

        return lax.fori_loop(0, C1 // 16, group, m)

    pmv[...] = jnp.full((16,), -1e30, jnp.float32)

    def chunk(i, c):
        b = i & 1
        for bb, (qrows, krows, qs, ks) in enumerate(
                ((qrows0, krows0, qs0, ks0), (qrows1, krows1, qs1, ks1))):
            @pl.when(b == bb)
            def _():
                pltpu.make_async_copy(
                    q_hbm.at[dsti.at[pl.ds(0, C1)]], qrows, qs).wait()
                pltpu.make_async_copy(
                    k_hbm.at[srci.at[pl.ds(0, C1)]], krows, ks).wait()
                pmv[...] = compute(i, qrows, krows, pmv[...])

                @pl.when(i + 2 < NCH1)
                def _():
                    issue(i + 2, qrows, krows, qs, ks)

        return c

    lax.fori_loop(0, NCH1, chunk, 0)
    pltpu.sync_copy(av, alpha_hbm.at[pl.ds(base0, EW1)])
    pltpu.sync_copy(pmv, pmax_hbm.at[pl.ds(wid * 16, 16)])


def _sc_pass1(q, k, src, dst):
    mesh = plsc.VectorSubcoreMesh(core_axis_name="c", subcore_axis_name="s")
    f = functools.partial(
        pl.kernel,
        mesh=mesh,
        out_type=[
            jax.ShapeDtypeStruct((E,), jnp.float32),
            jax.ShapeDtypeStruct((NW * 16,), jnp.float32),
        ],
        scratch_types=[
            pltpu.VMEM((EW1,), jnp.int32),
            pltpu.VMEM((EW1,), jnp.int32),
            pltpu.VMEM((C1, D_IN), jnp.float32),
            pltpu.VMEM((C1, D_IN), jnp.float32),
            pltpu.VMEM((C1, D_IN), jnp.float32),
            pltpu.VMEM((C1, D_IN), jnp.float32),
            pltpu.VMEM((EW1,), jnp.float32),
            pltpu.VMEM((16,), jnp.float32),
            pltpu.SemaphoreType.DMA,
            pltpu.SemaphoreType.DMA,
            pltpu.SemaphoreType.DMA,
            pltpu.SemaphoreType.DMA,
        ],
    )(_sc_pass1_body)
    return f(q, k, src, dst)


DEN_ROWS = 80


def _sc_den_body(dst_hbm, alpha_hbm, pmax_hbm, den0_hbm, den1_hbm,
                 den_sh, dsta, ava, didx0, didx1, md0, md1, pmr, ds0, ds1):
    cid = lax.axis_index("c")
    sid = lax.axis_index("s")
    wid = sid * NC + cid
    base0 = wid * EW1

    pltpu.sync_copy(pmax_hbm, pmr)

    def mrow(r, mv):
        return jnp.maximum(mv, pmr[pl.ds(16 * r, 16)])

    mv = lax.fori_loop(0, NW, mrow, jnp.full((16,), -1e30, jnp.float32))
    gmax = _lane_max(mv)

    pltpu.sync_copy(dst_hbm.at[pl.ds(base0, EW1)], dsta)
    pltpu.sync_copy(alpha_hbm.at[pl.ds(base0, EW1)], ava)

    def exloop(r, c):
        ava[pl.ds(16 * r, 16)] = jnp.exp(ava[pl.ds(16 * r, 16)] - gmax)
        return c

    lax.fori_loop(0, EW1 // 16, exloop, 0)

    def zmd(r, c):
        for j in range(8):
            md0[r, pl.ds(16 * j, 16)] = jnp.zeros((16,), jnp.float32)
            md1[r, pl.ds(16 * j, 16)] = jnp.zeros((16,), jnp.float32)
        return c

    lax.fori_loop(0, C1, zmd, 0)

    @pl.when(sid == 0)
    def _():
        pltpu.sync_copy(md0, den_sh)

    plsc.subcore_barrier()

    lanes = lax.iota(jnp.int32, 16)

    def chunk(j, c):
        b = j & 1
        for bb, (md, didx, dsem) in enumerate(
                ((md0, didx0, ds0), (md1, didx1, ds1))):
            @pl.when(b == bb)
            def _():
                @pl.when(j >= 2)
                def _():
                    pltpu.make_async_copy(md, den_sh.at[didx], dsem).wait()

                def grp(g, c2):
                    eb = j * C1 + g * 16
                    dv = dsta[pl.ds(eb, 16)]
                    exv = ava[pl.ds(eb, 16)]
                    didx[pl.ds(16 * g, 16)] = lax.shift_right_logical(dv, 7)
                    ebo = jnp.maximum(eb - 2 * C1, 0)
                    dvo = dsta[pl.ds(ebo, 16)]
                    for t in range(16):
                        e = 16 * g + t
                        md[e, pl.ds(dvo[t] & 112, 16)] = jnp.zeros(
                            (16,), jnp.float32)
                        d = dv[t]
                        md[e, pl.ds(d & 112, 16)] = jnp.where(
                            lanes == (d & 15), exv[t], 0.0)
                    return c2

                lax.fori_loop(0, C1 // 16, grp, 0)
                pltpu.async_copy(md, den_sh.at[didx], dsem, add=True)

        return c

    lax.fori_loop(0, NCH1, chunk, 0)
    pltpu.make_async_copy(md0, den_sh.at[didx0], ds0).wait()
    pltpu.make_async_copy(md1, den_sh.at[didx1], ds1).wait()
    plsc.subcore_barrier()

    @pl.when(sid == 0)
    def _():
        pltpu.sync_copy(den_sh, md0)

        @pl.when(cid == 0)
        def _():
            pltpu.sync_copy(md0, den0_hbm)

        @pl.when(cid == 1)
        def _():
            pltpu.sync_copy(md0, den1_hbm)


def _sc_den(dst, alpha, pmax):
    mesh = plsc.VectorSubcoreMesh(core_axis_name="c", subcore_axis_name="s")
    f = functools.partial(
        pl.kernel,
        mesh=mesh,
        out_type=[
            jax.ShapeDtypeStruct((DEN_ROWS, 128), jnp.float32),
            jax.ShapeDtypeStruct((DEN_ROWS, 128), jnp.float32),
        ],
        scratch_types=[
            pltpu.VMEM_SHARED((DEN_ROWS, 128), jnp.float32),
            pltpu.VMEM((EW1,), jnp.int32),
            pltpu.VMEM((EW1,), jnp.float32),
            pltpu.VMEM((C1,), jnp.int32),
            pltpu.VMEM((C1,), jnp.int32),
            pltpu.VMEM((C1, 128), jnp.float32),
            pltpu.VMEM((C1, 128), jnp.float32),
            pltpu.VMEM((NW * 16,), jnp.float32),
            pltpu.SemaphoreType.DMA,
            pltpu.SemaphoreType.DMA,
        ],
    )(_sc_den_body)
    return f(dst, alpha, pmax)


def _sc_pass2_body(v_hbm, src_hbm, dst_hbm, alpha_hbm, pmax_hbm,
                   out0_hbm, out1_hbm,
                   acc_sh,
                   srci0, srci1, dsti0, dsti1, scidx0, scidx1, vidx0, vidx1,
                   av0, av1, exv0, exv1, vrows0, vrows1, msg0, msg1, pmr,
                   is0, is1, gs0, gs1, ss0, ss1):
    cid = lax.axis_index("c")
    sid = lax.axis_index("s")

    pltpu.sync_copy(pmax_hbm, pmr)

    def mrow(r, mv):
        return jnp.maximum(mv, pmr[pl.ds(16 * r, 16)])

    mv = lax.fori_loop(0, NW, mrow, jnp.full((16,), -1e30, jnp.float32))
    gmax = _lane_max(mv)

    def zmsg(r, c):
        for j in range(8):
            msg0[r, pl.ds(16 * j, 16)] = jnp.zeros((16,), jnp.float32)
        return c

    lax.fori_loop(0, C2, zmsg, 0)

    def zchunk(t, c):
        ch = sid + t * NS

        @pl.when(ch < NDCHUNK)
        def _():
            pltpu.sync_copy(msg0, acc_sh.at[pl.ds(ch * DRAIN, DRAIN)])

        return c

    lax.fori_loop(0, NDROUND, zchunk, 0)
    plsc.subcore_barrier()

    ebase = sid * EW2
    bufs = ((srci0, dsti0, scidx0, vidx0, av0, exv0, vrows0, msg0, is0, gs0,
             ss0),
            (srci1, dsti1, scidx1, vidx1, av1, exv1, vrows1, msg1, is1, gs1,
             ss1))

    def vec16(srci, vidx, av, exv):
        def body(t, c):
            s = srci[pl.ds(16 * t, 16)]
            vidx[pl.ds(16 * t, 16)] = s * 2 + cid
            exv[pl.ds(16 * t, 16)] = jnp.exp(av[pl.ds(16 * t, 16)] - gmax)
            return c

        lax.fori_loop(0, C2 // 16, body, 0)

    def cp16(src_ref, dst_ref):
        def body(t, c):
            dst_ref[pl.ds(16 * t, 16)] = src_ref[pl.ds(16 * t, 16)]
            return c

        lax.fori_loop(0, C2 // 16, body, 0)

    def issue_idx(i, srci, dsti, av, isem):
        base = ebase + i * C2
        pltpu.async_copy(src_hbm.at[pl.ds(base, C2)], srci, isem)
        pltpu.async_copy(dst_hbm.at[pl.ds(base, C2)], dsti, isem)
        pltpu.async_copy(alpha_hbm.at[pl.ds(base, C2)], av, isem)

    def wait_idx(srci, dsti, av, isem):
        pltpu.make_async_copy(src_hbm.at[pl.ds(0, C2)], srci, isem).wait()
        pltpu.make_async_copy(dst_hbm.at[pl.ds(0, C2)], dsti, isem).wait()
        pltpu.make_async_copy(alpha_hbm.at[pl.ds(0, C2)], av, isem).wait()

    pltpu.sync_copy(src_hbm.at[pl.ds(ebase, C2)], srci0)
    pltpu.sync_copy(dst_hbm.at[pl.ds(ebase, C2)], dsti0)
    pltpu.sync_copy(alpha_hbm.at[pl.ds(ebase, C2)], av0)
    vec16(srci0, vidx0, av0, exv0)
    pltpu.async_copy(v_hbm.at[vidx0], vrows0, gs0)
    issue_idx(1, srci1, dsti1, av1, is1)

    def chunk(i, c):
        b = i & 1
        for bb in (0, 1):
            (srci, dsti, scidx, vidx, av, exv, vrows, msg, isem, gsem,
             ssem) = bufs[bb]
            (srcn, dstn, scidxn, vidxn, avn, exvn, vrowsn, msgn, isemn,
             gsemn, ssemn) = bufs[1 - bb]

            @pl.when(b == bb)
            def _():
                @pl.when(i + 1 < NCH2)
                def _():
                    wait_idx(srcn, dstn, avn, isemn)
                    vec16(srcn, vidxn, avn, exvn)
                    pltpu.async_copy(v_hbm.at[vidxn], vrowsn, gsemn)

                cp16(dsti, scidx)

                @pl.when(i + 2 < NCH2)
                def _():
                    issue_idx(i + 2, srci, dsti, av, isem)

                pltpu.make_async_copy(v_hbm.at[vidx], vrows, gsem).wait()

                def group(g, c2):
                    exq = exv[pl.ds(g * 16, 16)]
                    for t in range(16):
                        e = g * 16 + t
                        ex = exq[t]
                        for j in range(8):
                            msg[e, pl.ds(16 * j, 16)] = (
                                vrows[e, pl.ds(16 * j, 16)] * ex)
                    return c2

                lax.fori_loop(0, C2 // 16, group, 0)
                pltpu.sync_copy(msg, acc_sh.at[scidx], add=True)

        return c

    lax.fori_loop(0, NCH2, chunk, 0)
    plsc.subcore_barrier()

    def drain(t, c):
        ch = sid + t * NS

        @pl.when(ch < NDCHUNK)
        def _():
            r = ch * DRAIN
            pltpu.sync_copy(acc_sh.at[pl.ds(r, DRAIN)], msg0)

            @pl.when(cid == 0)
            def _():
                pltpu.sync_copy(msg0, out0_hbm.at[pl.ds(r, DRAIN)])

            @pl.when(cid == 1)
            def _():
                pltpu.sync_copy(msg0, out1_hbm.at[pl.ds(r, DRAIN)])

        return c

    lax.fori_loop(0, NDROUND, drain, 0)


def _sc_pass2(v2, src, dst, alpha, pmax):
    mesh = plsc.VectorSubcoreMesh(core_axis_name="c", subcore_axis_name="s")
    f = functools.partial(
        pl.kernel,
        mesh=mesh,
        out_type=[
            jax.ShapeDtypeStruct((N_MOVIE, ACC_W), jnp.float32),
            jax.ShapeDtypeStruct((N_MOVIE, ACC_W), jnp.float32),
        ],
        scratch_types=[
            pltpu.VMEM_SHARED((N_MOVIE, ACC_W), jnp.float32),
            pltpu.VMEM((C2,), jnp.int32),
            pltpu.VMEM((C2,), jnp.int32),
            pltpu.VMEM((C2,), jnp.int32),
            pltpu.VMEM((C2,), jnp.int32),
            pltpu.VMEM((C2,), jnp.int32),
            pltpu.VMEM((C2,), jnp.int32),
            pltpu.VMEM((C2,), jnp.int32),
            pltpu.VMEM((C2,), jnp.int32),
            pltpu.VMEM((C2,), jnp.float32),
            pltpu.VMEM((C2,), jnp.float32),
            pltpu.VMEM((C2,), jnp.float32),
            pltpu.VMEM((C2,), jnp.float32),
            pltpu.VMEM((C2, 128), jnp.float32),
            pltpu.VMEM((C2, 128), jnp.float32),
            pltpu.VMEM((C2, ACC_W), jnp.float32),
            pltpu.VMEM((C2, ACC_W), jnp.float32),
            pltpu.VMEM((NW * 16,), jnp.float32),
            pltpu.SemaphoreType.DMA,
            pltpu.SemaphoreType.DMA,
            pltpu.SemaphoreType.DMA,
            pltpu.SemaphoreType.DMA,
            pltpu.SemaphoreType.DMA,
            pltpu.SemaphoreType.DMA,
        ],
    )(_sc_pass2_body)
    return f(v2, src, dst, alpha, pmax)


def _tc_final_body(o0_ref, o1_ref, d0_ref, d1_ref, wa_ref, ba_ref, wlin_ref,
                   blin_ref, y_ref, walf, balf):
    @pl.when(pl.program_id(0) == 0)
    def _fold():
        walf[...] = jnp.dot(wa_ref[...], wlin_ref[...],
                            preferred_element_type=jnp.float32)
        balf[...] = jnp.dot(ba_ref[...], wlin_ref[...],
                            preferred_element_type=jnp.float32) + blin_ref[...]

    agg = jnp.concatenate([o0_ref[...], o1_ref[...]], axis=1)
    den = d0_ref[...] + d1_ref[...] + 1e-16
    h = jax.nn.gelu(agg / den)
    y_ref[...] = jnp.dot(h, walf[...],
                         preferred_element_type=jnp.float32) + balf[...]


def _tc_final(o0, o1, d0, d1, wa, ba, wlin, blin):
    R = 1000
    grid = (N_MOVIE // R,)
    return pl.pallas_call(
        _tc_final_body,
        grid=grid,
        in_specs=[
            pl.BlockSpec((R, ACC_W), lambda i: (i, 0)),
            pl.BlockSpec((R, ACC_W), lambda i: (i, 0)),
            pl.BlockSpec((R, 1), lambda i: (i, 0)),
            pl.BlockSpec((R, 1), lambda i: (i, 0)),
            pl.BlockSpec((D_HID, D_HID), lambda i: (0, 0)),
            pl.BlockSpec((1, D_HID), lambda i: (0, 0)),
            pl.BlockSpec((D_HID, N_CLS), lambda i: (0, 0)),
            pl.BlockSpec((1, N_CLS), lambda i: (0, 0)),
        ],
        out_specs=pl.BlockSpec((R, N_CLS), lambda i: (i, 0)),
        out_shape=jax.ShapeDtypeStruct((N_MOVIE, N_CLS), jnp.float32),
        scratch_shapes=[
            pltpu.VMEM((D_HID, N_CLS), jnp.float32),
            pltpu.VMEM((1, N_CLS), jnp.float32),
        ],
        compiler_params=pltpu.CompilerParams(
            dimension_semantics=("arbitrary",)),
    )(o0, o1, d0, d1, wa, ba, wlin, blin)


def kernel(x_movie, x_author, Wk, bk, Wq, bq, Wv, bv, a_rel, m_rel, p_rel,
           Wa, ba, Wlin, blin, edge_index):
    ei = edge_index.astype(jnp.int32)
    src = ei[0]
    dst = ei[1]
    arel = a_rel.reshape(D_HID, D_HID)
    mrel = m_rel.reshape(D_HID, D_HID)

    s = p_rel.reshape(-1)[0].astype(jnp.float32) * (1.0 / 16.0)
    w2 = Wk @ arel
    g = (s * Wq) @ w2.T
    w = w2 @ (bq * s)
    dsolve = jnp.linalg.solve(g.T, w)
    delta = jnp.where(jnp.any(w != 0), dsolve, jnp.zeros_like(w))

    q, k, v = _tc_proj(x_movie, x_author, g, delta.reshape(1, D_IN),
                       Wv, bv.reshape(1, D_HID), mrel)

    alpha, pmax = _sc_pass1(q, k, src, dst)
    den0, den1 = _sc_den(dst, alpha, pmax)

    v2 = v.reshape(2 * N_MOVIE, 128)
    o0, o1 = _sc_pass2(v2, src, dst, alpha, pmax)

    d0 = den0.reshape(DEN_ROWS * 128)[:N_MOVIE].reshape(N_MOVIE, 1)
    d1 = den1.reshape(DEN_ROWS * 128)[:N_MOVIE].reshape(N_MOVIE, 1)
    return _tc_final(o0, o1, d0, d1,
                     Wa, ba.reshape(1, D_HID),
                     Wlin, blin.reshape(1, N_CLS))
